# Initial kernel scaffold; baseline (speedup 1.0000x reference)
import jax
import jax.numpy as jnp
from jax.experimental import pallas as pl


def kernel(x, tok_table):
    raise NotImplementedError("write your pallas kernel here")


# speedup vs baseline: 1.0278x; 1.0278x over previous
import functools

import jax
import jax.numpy as jnp
from jax import lax
from jax.experimental import pallas as pl
from jax.experimental.pallas import tpu as pltpu
from jax.experimental.pallas import tpu_sc as plsc

_VOCAB = 1000000
_D = 128
_SEQ = 8192

_NC = 2
_NS = 16
_L = 16
_NW = _NC * _NS
_B_PER_W = _SEQ // _NW


def _pos_table(d_model, max_len):
    pos = jnp.arange(max_len, dtype=jnp.float32)[:, None]
    _2i = jnp.arange(0, d_model, 2, dtype=jnp.float32)
    angle = pos / (10000.0 ** (_2i / d_model))
    table = jnp.zeros((max_len, d_model), dtype=jnp.float32)
    table = table.at[:, 0::2].set(jnp.sin(angle))
    table = table.at[:, 1::2].set(jnp.cos(angle))
    return table


def _body(tok_hbm, idx_hbm, pos_hbm, out_hbm, idx_v, rows_v, pos_v, gsem):
    wid = lax.axis_index("s") * _NC + lax.axis_index("c")
    base = wid * _B_PER_W

    pltpu.sync_copy(idx_hbm.at[pl.ds(base, _B_PER_W)], idx_v)
    gather = pltpu.async_copy(tok_hbm.at[idx_v], rows_v, gsem)
    pltpu.sync_copy(pos_hbm.at[pl.ds(base, _B_PER_W)], pos_v)
    gather.wait()

    def add_row(r, carry):
        for c in range(_D // _L):
            sl = pl.ds(c * _L, _L)
            plsc.addupdate(rows_v.at[r, sl], pos_v[r, sl])
        return carry

    lax.fori_loop(0, _B_PER_W, add_row, 0, unroll=2)

    pltpu.sync_copy(rows_v, out_hbm.at[pl.ds(base, _B_PER_W)])


@jax.jit
def _embed(x, tok_table, pos):
    mesh = plsc.VectorSubcoreMesh(
        core_axis_name="c", subcore_axis_name="s",
        num_cores=_NC, num_subcores=_NS)
    return pl.kernel(
        _body,
        out_type=jax.ShapeDtypeStruct((_SEQ, _D), jnp.float32),
        mesh=mesh,
        scratch_types=[
            pltpu.VMEM((_B_PER_W,), jnp.int32),
            pltpu.VMEM((_B_PER_W, _D), jnp.float32),
            pltpu.VMEM((_B_PER_W, _D), jnp.float32),
            pltpu.SemaphoreType.DMA,
        ],
    )(tok_table, x, pos)


def kernel(x, tok_table):
    pos = _pos_table(_D, _SEQ)
    return _embed(x.astype(jnp.int32), tok_table, pos)


# speedup vs baseline: 1.6838x; 1.6383x over previous
import jax
import jax.numpy as jnp
import numpy as np
from jax import lax
from jax.experimental import pallas as pl
from jax.experimental.pallas import tpu as pltpu
from jax.experimental.pallas import tpu_sc as plsc

_VOCAB = 1000000
_D = 128
_SEQ = 8192

_NC = 2
_NS = 16
_L = 16
_NW = _NC * _NS
_B_PER_W = _SEQ // _NW


def _pos_table(d_model, max_len):
    pos = np.arange(max_len, dtype=np.float64)[:, None]
    _2i = np.arange(0, d_model, 2, dtype=np.float64)
    angle = (pos.astype(np.float32) /
             (10000.0 ** (_2i / d_model)).astype(np.float32))
    table = np.zeros((max_len, d_model), dtype=np.float32)
    table[:, 0::2] = np.sin(angle, dtype=np.float32)
    table[:, 1::2] = np.cos(angle, dtype=np.float32)
    return table


_POS = jnp.asarray(_pos_table(_D, _SEQ))


def _body(tok_hbm, idx_hbm, pos_hbm, out_hbm, idx_v, rows_v, pos_v, gsem):
    wid = lax.axis_index("s") * _NC + lax.axis_index("c")
    base = wid * _B_PER_W

    pltpu.sync_copy(idx_hbm.at[pl.ds(base, _B_PER_W)], idx_v)
    gather = pltpu.async_copy(tok_hbm.at[idx_v], rows_v, gsem)
    pltpu.sync_copy(pos_hbm.at[pl.ds(base, _B_PER_W)], pos_v)
    gather.wait()

    def add_row(r, carry):
        for c in range(_D // _L):
            sl = pl.ds(c * _L, _L)
            plsc.addupdate(rows_v.at[r, sl], pos_v[r, sl])
        return carry

    lax.fori_loop(0, _B_PER_W, add_row, 0, unroll=2)

    pltpu.sync_copy(rows_v, out_hbm.at[pl.ds(base, _B_PER_W)])


def _embed(x, tok_table, pos):
    mesh = plsc.VectorSubcoreMesh(
        core_axis_name="c", subcore_axis_name="s",
        num_cores=_NC, num_subcores=_NS)
    return pl.kernel(
        _body,
        out_type=jax.ShapeDtypeStruct((_SEQ, _D), jnp.float32),
        mesh=mesh,
        scratch_types=[
            pltpu.VMEM((_B_PER_W,), jnp.int32),
            pltpu.VMEM((_B_PER_W, _D), jnp.float32),
            pltpu.VMEM((_B_PER_W, _D), jnp.float32),
            pltpu.SemaphoreType.DMA,
        ],
    )(tok_table, x, pos)


def kernel(x, tok_table):
    return _embed(x.astype(jnp.int32), tok_table, _POS)


# speedup vs baseline: 1.6916x; 1.0046x over previous
import jax
import jax.numpy as jnp
import numpy as np
from jax import lax
from jax.experimental import pallas as pl
from jax.experimental.pallas import tpu as pltpu
from jax.experimental.pallas import tpu_sc as plsc

_VOCAB = 1000000
_D = 128
_SEQ = 8192

_NC = 2
_NS = 16
_L = 16
_NW = _NC * _NS
_B_PER_W = _SEQ // _NW
_K = 2
_R = _B_PER_W // _K


def _pos_table(d_model, max_len):
    pos = np.arange(max_len, dtype=np.float32)[:, None]
    _2i = np.arange(0, d_model, 2, dtype=np.float32)
    angle = pos / np.float32(10000.0) ** (_2i / np.float32(d_model))
    table = np.zeros((max_len, d_model), dtype=np.float32)
    table[:, 0::2] = np.sin(angle)
    table[:, 1::2] = np.cos(angle)
    return table


_POS_FLAT = jnp.asarray(_pos_table(_D, _SEQ).reshape(-1))


def _body(tok_hbm, idx_hbm, pos_hbm, out_hbm,
          idx_v, rows_v, pos_v, gsem0, gsem1, psem, wsem):
    wid = lax.axis_index("s") * _NC + lax.axis_index("c")
    base = wid * _B_PER_W

    pltpu.sync_copy(idx_hbm.at[pl.ds(base, _B_PER_W)], idx_v)
    g0 = pltpu.async_copy(tok_hbm.at[idx_v.at[pl.ds(0, _R)]],
                          rows_v.at[0], gsem0)
    g1 = pltpu.async_copy(tok_hbm.at[idx_v.at[pl.ds(_R, _R)]],
                          rows_v.at[1], gsem1)
    pg = pltpu.async_copy(pos_hbm.at[pl.ds(base * _D, _B_PER_W * _D)],
                          pos_v, psem)
    pg.wait()
    g0.wait()

    def add_rows(k):
        def add_row(r, carry):
            off = (k * _R + r) * _D
            for c in range(_D // _L):
                plsc.addupdate(rows_v.at[k, r, pl.ds(c * _L, _L)],
                               pos_v[pl.ds(off + c * _L, _L)])
            return carry
        lax.fori_loop(0, _R, add_row, 0, unroll=2)

    add_rows(0)
    w0 = pltpu.async_copy(rows_v.at[0],
                          out_hbm.at[pl.ds(base, _R)], wsem)
    g1.wait()
    add_rows(1)
    w1 = pltpu.async_copy(rows_v.at[1],
                          out_hbm.at[pl.ds(base + _R, _R)], wsem)
    w0.wait()
    w1.wait()


def _embed(x, tok_table, pos):
    mesh = plsc.VectorSubcoreMesh(
        core_axis_name="c", subcore_axis_name="s",
        num_cores=_NC, num_subcores=_NS)
    return pl.kernel(
        _body,
        out_type=jax.ShapeDtypeStruct((_SEQ, _D), jnp.float32),
        mesh=mesh,
        scratch_types=[
            pltpu.VMEM((_B_PER_W,), jnp.int32),
            pltpu.VMEM((_K, _R, _D), jnp.float32),
            pltpu.VMEM((_B_PER_W * _D,), jnp.float32),
            pltpu.SemaphoreType.DMA,
            pltpu.SemaphoreType.DMA,
            pltpu.SemaphoreType.DMA,
            pltpu.SemaphoreType.DMA,
        ],
    )(tok_table, x, pos)


def kernel(x, tok_table):
    return _embed(x.astype(jnp.int32), tok_table, _POS_FLAT)


# speedup vs baseline: 1.8373x; 1.0861x over previous
import jax
import jax.numpy as jnp
import numpy as np
from jax import lax
from jax.experimental import pallas as pl
from jax.experimental.pallas import tpu as pltpu
from jax.experimental.pallas import tpu_sc as plsc

_VOCAB = 1000000
_D = 128
_SEQ = 8192

_NC = 2
_NS = 16
_L = 16
_NW = _NC * _NS
_B_PER_W = _SEQ // _NW
_K = 2
_R = _B_PER_W // _K
_NCH = _D // _L


def _pos_tables():
    j = np.arange(_D)
    w = (10000.0 ** (-2.0 * (j // 2) / _D)).astype(np.float64)
    phase = np.where(j % 2 == 0, 0.0, np.pi / 2.0)
    starts = np.arange(_SEQ // _R, dtype=np.float64) * _R
    ang0 = starts[:, None] * w[None, :] + phase[None, :]
    init = np.stack([np.sin(ang0), np.cos(ang0)], axis=1)
    rot = np.stack([np.cos(w), np.sin(w)], axis=0)
    return (init.astype(np.float32).reshape(-1),
            rot.astype(np.float32).reshape(-1))


_INIT_FLAT, _ROT_FLAT = (jnp.asarray(t) for t in _pos_tables())


def _body(tok_hbm, idx_hbm, init_hbm, rot_hbm, out_hbm,
          idx_v, rows_v, st_v, rot_v, gsem0, gsem1, wsem):
    wid = lax.axis_index("s") * _NC + lax.axis_index("c")
    base = wid * _B_PER_W

    pltpu.sync_copy(idx_hbm.at[pl.ds(base, _B_PER_W)], idx_v)
    g0 = pltpu.async_copy(tok_hbm.at[idx_v.at[pl.ds(0, _R)]],
                          rows_v.at[0], gsem0)
    g1 = pltpu.async_copy(tok_hbm.at[idx_v.at[pl.ds(_R, _R)]],
                          rows_v.at[1], gsem1)
    pltpu.sync_copy(init_hbm.at[pl.ds(wid * (_K * 2 * _D), _K * 2 * _D)],
                    st_v)
    pltpu.sync_copy(rot_hbm, rot_v)
    g0.wait()

    def add_rows(k):
        cw = [rot_v[pl.ds(c * _L, _L)] for c in range(_NCH)]
        sw = [rot_v[pl.ds(_D + c * _L, _L)] for c in range(_NCH)]
        v0 = [st_v[pl.ds(k * 2 * _D + c * _L, _L)] for c in range(_NCH)]
        u0 = [st_v[pl.ds((k * 2 + 1) * _D + c * _L, _L)]
              for c in range(_NCH)]

        def add_row(r, state):
            v, u = state
            nv, nu = [], []
            for c in range(_NCH):
                plsc.addupdate(rows_v.at[k, r, pl.ds(c * _L, _L)], v[c])
                nv.append(v[c] * cw[c] + u[c] * sw[c])
                nu.append(u[c] * cw[c] - v[c] * sw[c])
            return tuple(nv), tuple(nu)

        lax.fori_loop(0, _R, add_row, (tuple(v0), tuple(u0)), unroll=2)

    add_rows(0)
    w0 = pltpu.async_copy(rows_v.at[0],
                          out_hbm.at[pl.ds(base, _R)], wsem)
    g1.wait()
    add_rows(1)
    w1 = pltpu.async_copy(rows_v.at[1],
                          out_hbm.at[pl.ds(base + _R, _R)], wsem)
    w0.wait()
    w1.wait()


def _embed(x, tok_table, init, rot):
    mesh = plsc.VectorSubcoreMesh(
        core_axis_name="c", subcore_axis_name="s",
        num_cores=_NC, num_subcores=_NS)
    return pl.kernel(
        _body,
        out_type=jax.ShapeDtypeStruct((_SEQ, _D), jnp.float32),
        mesh=mesh,
        scratch_types=[
            pltpu.VMEM((_B_PER_W,), jnp.int32),
            pltpu.VMEM((_K, _R, _D), jnp.float32),
            pltpu.VMEM((_K * 2 * _D,), jnp.float32),
            pltpu.VMEM((2 * _D,), jnp.float32),
            pltpu.SemaphoreType.DMA,
            pltpu.SemaphoreType.DMA,
            pltpu.SemaphoreType.DMA,
        ],
    )(tok_table, x, init, rot)


def kernel(x, tok_table):
    return _embed(x.astype(jnp.int32), tok_table, _INIT_FLAT, _ROT_FLAT)


# speedup vs baseline: 1.8689x; 1.0172x over previous
import jax
import jax.numpy as jnp
import numpy as np
from jax import lax
from jax.experimental import pallas as pl
from jax.experimental.pallas import tpu as pltpu
from jax.experimental.pallas import tpu_sc as plsc

_VOCAB = 1000000
_D = 128
_SEQ = 8192

_NC = 2
_NS = 16
_L = 16
_NW = _NC * _NS
_B_PER_W = _SEQ // _NW
_K = 2
_R = _B_PER_W // _K
_NCH = _D // _L

_INIT_LEN = (_SEQ // _R) * 2 * _D
_ROT_OFF = _SEQ + _INIT_LEN
_ST_LEN = _K * 2 * _D


def _packed_consts():
    j = np.arange(_D)
    w = (10000.0 ** (-2.0 * (j // 2) / _D)).astype(np.float64)
    phase = np.where(j % 2 == 0, 0.0, np.pi / 2.0)
    starts = np.arange(_SEQ // _R, dtype=np.float64) * _R
    ang0 = starts[:, None] * w[None, :] + phase[None, :]
    init = np.stack([np.sin(ang0), np.cos(ang0)], axis=1)
    rot = np.stack([np.cos(w), np.sin(w)], axis=0)
    packed = np.concatenate([init.astype(np.float32).reshape(-1),
                             rot.astype(np.float32).reshape(-1)])
    return packed.view(np.int32)


_CONSTS_I32 = _packed_consts()


def _f32(chunk):
    return lax.bitcast_convert_type(chunk, jnp.float32)


def _body(tok_hbm, xx_hbm, out_hbm, idx_v, rows_v, st_v, rot_v,
          gsem0, gsem1, wsem):
    wid = lax.axis_index("s") * _NC + lax.axis_index("c")
    base = wid * _B_PER_W

    pltpu.sync_copy(xx_hbm.at[pl.ds(base, _B_PER_W)], idx_v)
    g0 = pltpu.async_copy(tok_hbm.at[idx_v.at[pl.ds(0, _R)]],
                          rows_v.at[0], gsem0)
    g1 = pltpu.async_copy(tok_hbm.at[idx_v.at[pl.ds(_R, _R)]],
                          rows_v.at[1], gsem1)
    pltpu.sync_copy(xx_hbm.at[pl.ds(_SEQ + wid * _ST_LEN, _ST_LEN)], st_v)
    pltpu.sync_copy(xx_hbm.at[pl.ds(_ROT_OFF, 2 * _D)], rot_v)
    g0.wait()

    def add_rows(k):
        cw = [_f32(rot_v[pl.ds(c * _L, _L)]) for c in range(_NCH)]
        sw = [_f32(rot_v[pl.ds(_D + c * _L, _L)]) for c in range(_NCH)]
        v0 = [_f32(st_v[pl.ds(k * 2 * _D + c * _L, _L)])
              for c in range(_NCH)]
        u0 = [_f32(st_v[pl.ds((k * 2 + 1) * _D + c * _L, _L)])
              for c in range(_NCH)]

        def add_row(r, state):
            v, u = state
            nv, nu = [], []
            for c in range(_NCH):
                plsc.addupdate(rows_v.at[k, r, pl.ds(c * _L, _L)], v[c])
                nv.append(v[c] * cw[c] + u[c] * sw[c])
                nu.append(u[c] * cw[c] - v[c] * sw[c])
            return tuple(nv), tuple(nu)

        lax.fori_loop(0, _R, add_row, (tuple(v0), tuple(u0)), unroll=2)

    add_rows(0)
    w0 = pltpu.async_copy(rows_v.at[0],
                          out_hbm.at[pl.ds(base, _R)], wsem)
    g1.wait()
    add_rows(1)
    w1 = pltpu.async_copy(rows_v.at[1],
                          out_hbm.at[pl.ds(base + _R, _R)], wsem)
    w0.wait()
    w1.wait()


def _embed(xx, tok_table):
    mesh = plsc.VectorSubcoreMesh(
        core_axis_name="c", subcore_axis_name="s",
        num_cores=_NC, num_subcores=_NS)
    return pl.kernel(
        _body,
        out_type=jax.ShapeDtypeStruct((_SEQ, _D), jnp.float32),
        mesh=mesh,
        scratch_types=[
            pltpu.VMEM((_B_PER_W,), jnp.int32),
            pltpu.VMEM((_K, _R, _D), jnp.float32),
            pltpu.VMEM((_ST_LEN,), jnp.int32),
            pltpu.VMEM((2 * _D,), jnp.int32),
            pltpu.SemaphoreType.DMA,
            pltpu.SemaphoreType.DMA,
            pltpu.SemaphoreType.DMA,
        ],
    )(tok_table, xx)


def kernel(x, tok_table):
    xx = jnp.concatenate([x.astype(jnp.int32), _CONSTS_I32])
    return _embed(xx, tok_table)
